# baseline (speedup 1.0000x reference)
import functools

import jax
import jax.numpy as jnp
from jax import lax
from jax.experimental import pallas as pl
from jax.experimental.pallas import tpu as pltpu
from jax.experimental.pallas import tpu_sc as plsc

N_NODES = 100000
MEM_DIM = 128
MSG_DIM = 256
B = 4096

_NC = 2
_NS = 16
_NW = _NC * _NS
_CHUNK = B // _NW

_CP_CHUNK = 3200
_CP_LAST = N_NODES - _CP_CHUNK * (_NW - 1)


def _sc_mesh():
    return plsc.VectorSubcoreMesh(
        core_axis_name="c", subcore_axis_name="s", num_cores=_NC, num_subcores=_NS
    )


def _worker_id():
    return lax.axis_index("s") * _NC + lax.axis_index("c")


def _sc_gather(mem, idx):
    @functools.partial(
        pl.kernel,
        out_type=jax.ShapeDtypeStruct((B, MEM_DIM), jnp.float32),
        mesh=_sc_mesh(),
        scratch_types=[
            pltpu.VMEM((_CHUNK,), jnp.int32),
            pltpu.VMEM((_CHUNK, MEM_DIM), jnp.float32),
            pltpu.SemaphoreType.DMA,
        ],
    )
    def gk(mem_hbm, idx_hbm, out_hbm, idx_v, rows_v, sem):
        base = _worker_id() * _CHUNK
        pltpu.sync_copy(idx_hbm.at[pl.ds(base, _CHUNK)], idx_v)
        pltpu.async_copy(mem_hbm.at[idx_v], rows_v, sem).wait()
        pltpu.sync_copy(rows_v, out_hbm.at[pl.ds(base, _CHUNK)])

    return gk(mem, idx)


def _sc_copy(mem, lu, mem_ref, lu_ref):
    @functools.partial(
        pl.kernel,
        out_type=(),
        mesh=_sc_mesh(),
        scratch_types=[pltpu.VMEM((_CP_CHUNK,), jnp.float32)],
    )
    def ck(mem_hbm, lu_hbm, outmem_hbm, outlu_hbm, lu_v):
        wid = _worker_id()
        r0 = wid * _CP_CHUNK

        @pl.when(wid < _NW - 1)
        def _():
            pltpu.sync_copy(mem_hbm.at[pl.ds(r0, _CP_CHUNK)], outmem_hbm.at[pl.ds(r0, _CP_CHUNK)])
            pltpu.sync_copy(lu_hbm.at[pl.ds(r0, _CP_CHUNK)], lu_v)
            pltpu.sync_copy(lu_v, outlu_hbm.at[pl.ds(r0, _CP_CHUNK)])

        @pl.when(wid == _NW - 1)
        def _():
            last = (_NW - 1) * _CP_CHUNK
            pltpu.sync_copy(mem_hbm.at[pl.ds(last, _CP_LAST)], outmem_hbm.at[pl.ds(last, _CP_LAST)])
            pltpu.sync_copy(lu_hbm.at[pl.ds(last, _CP_LAST)], lu_v.at[pl.ds(0, _CP_LAST)])
            pltpu.sync_copy(lu_v.at[pl.ds(0, _CP_LAST)], outlu_hbm.at[pl.ds(last, _CP_LAST)])

    ck(mem, lu, mem_ref, lu_ref)


_GRU_BLK = 512


def _gru_body(x_ref, h_ref, wih_ref, whh_ref, bih_ref, bhh_ref, out_ref):
    x = x_ref[...]
    h32 = h_ref[...]
    h = h32.astype(jnp.bfloat16)
    gi = jnp.dot(x, wih_ref[...], preferred_element_type=jnp.float32) + bih_ref[...]
    gh = jnp.dot(h, whh_ref[...], preferred_element_type=jnp.float32) + bhh_ref[...]
    i_r, i_z, i_n = gi[:, :MEM_DIM], gi[:, MEM_DIM : 2 * MEM_DIM], gi[:, 2 * MEM_DIM :]
    h_r, h_z, h_n = gh[:, :MEM_DIM], gh[:, MEM_DIM : 2 * MEM_DIM], gh[:, 2 * MEM_DIM :]
    r = jax.nn.sigmoid(i_r + h_r)
    z = jax.nn.sigmoid(i_z + h_z)
    n = jnp.tanh(i_n + r * h_n)
    out_ref[...] = n + z * (h32 - n)


def _tc_gru(x, h, W_ih, W_hh, b_ih, b_hh):
    x16 = x.astype(jnp.bfloat16)
    wih_t = W_ih.T.astype(jnp.bfloat16)
    whh_t = W_hh.T.astype(jnp.bfloat16)
    bih = b_ih.reshape(1, -1)
    bhh = b_hh.reshape(1, -1)
    grid = B // _GRU_BLK
    return pl.pallas_call(
        _gru_body,
        grid=(grid,),
        in_specs=[
            pl.BlockSpec((_GRU_BLK, MSG_DIM), lambda i: (i, 0)),
            pl.BlockSpec((_GRU_BLK, MEM_DIM), lambda i: (i, 0)),
            pl.BlockSpec((MSG_DIM, 3 * MEM_DIM), lambda i: (0, 0)),
            pl.BlockSpec((MEM_DIM, 3 * MEM_DIM), lambda i: (0, 0)),
            pl.BlockSpec((1, 3 * MEM_DIM), lambda i: (0, 0)),
            pl.BlockSpec((1, 3 * MEM_DIM), lambda i: (0, 0)),
        ],
        out_specs=pl.BlockSpec((_GRU_BLK, MEM_DIM), lambda i: (i, 0)),
        out_shape=jax.ShapeDtypeStruct((B, MEM_DIM), jnp.float32),
    )(x16, h, wih_t, whh_t, bih, bhh)


_JL_CHUNK = 512


def _jlast_body(nlane_ref, ncol_ref, out_ref):
    ni = nlane_ref[0]
    blk = pl.program_id(0)
    c0 = blk // (_JL_CHUNK // 128)

    def step(c, best):
        j0 = pl.multiple_of(c * _JL_CHUNK, _JL_CHUNK)
        nj = ncol_ref[pl.ds(j0, _JL_CHUNK), :]
        njb = jnp.broadcast_to(nj, (_JL_CHUNK, 128))
        jv = lax.broadcasted_iota(jnp.int32, (_JL_CHUNK, 128), 0) + j0
        m = jnp.where(njb == ni, jv, -1)
        return jnp.maximum(best, jnp.max(m, axis=0, keepdims=True))

    best = jnp.full((1, 128), -1, jnp.int32)
    out_ref[0] = lax.fori_loop(c0, B // _JL_CHUNK, step, best)


def _tc_jlast(nodes):
    nlane = nodes.reshape(B // 128, 1, 128)
    ncol = nodes.reshape(B, 1)
    out = pl.pallas_call(
        _jlast_body,
        grid=(B // 128,),
        in_specs=[
            pl.BlockSpec((1, 1, 128), lambda i: (i, 0, 0)),
            pl.BlockSpec((B, 1), lambda i: (0, 0)),
        ],
        out_specs=pl.BlockSpec((1, 1, 128), lambda i: (i, 0, 0)),
        out_shape=jax.ShapeDtypeStruct((B // 128, 1, 128), jnp.int32),
    )(nlane, ncol)
    return out.reshape(B)


def _sc_scatter(new_h, j_last, idx, ts, mem_ref, lu_ref):
    @functools.partial(
        pl.kernel,
        out_type=(),
        mesh=_sc_mesh(),
        scratch_types=[
            pltpu.VMEM((_CHUNK,), jnp.int32),
            pltpu.VMEM((_CHUNK,), jnp.int32),
            pltpu.VMEM((_CHUNK, MEM_DIM), jnp.float32),
            pltpu.VMEM((_CHUNK,), jnp.float32),
            pltpu.SemaphoreType.DMA,
            pltpu.SemaphoreType.DMA,
        ],
    )
    def sk(newh_hbm, jl_hbm, idx_hbm, ts_hbm, outmem_hbm, outlu_hbm,
           jl_v, idx_v, rows_v, ts_v, sem1, sem2):
        base = _worker_id() * _CHUNK
        pltpu.sync_copy(jl_hbm.at[pl.ds(base, _CHUNK)], jl_v)
        pltpu.sync_copy(idx_hbm.at[pl.ds(base, _CHUNK)], idx_v)
        g1 = pltpu.async_copy(newh_hbm.at[jl_v], rows_v, sem1)
        g2 = pltpu.async_copy(ts_hbm.at[jl_v], ts_v, sem2)
        g1.wait()
        g2.wait()
        s1 = pltpu.async_copy(rows_v, outmem_hbm.at[idx_v], sem1)
        s2 = pltpu.async_copy(ts_v, outlu_hbm.at[idx_v], sem2)
        s1.wait()
        s2.wait()

    sk(new_h, j_last, idx, ts, mem_ref, lu_ref)


def kernel(memory_tensor, last_update, unique_nodes, unique_messages, unique_ts, W_ih, W_hh, b_ih, b_hh):
    h = _sc_gather(memory_tensor, unique_nodes)
    mem_ref = jax.new_ref(lax.empty((N_NODES, MEM_DIM), jnp.float32))
    lu_ref = jax.new_ref(lax.empty((N_NODES,), jnp.float32))
    _sc_copy(memory_tensor, last_update, mem_ref, lu_ref)
    j_last = _tc_jlast(unique_nodes)
    new_h = _tc_gru(unique_messages, h, W_ih, W_hh, b_ih, b_hh)
    _sc_scatter(new_h, j_last, unique_nodes, unique_ts, mem_ref, lu_ref)
    return mem_ref[...], lu_ref[...]


# speedup vs baseline: 18.4905x; 18.4905x over previous
import functools

import jax
import jax.numpy as jnp
from jax import lax
from jax.experimental import pallas as pl
from jax.experimental.pallas import tpu as pltpu
from jax.experimental.pallas import tpu_sc as plsc

N_NODES = 100000
MEM_DIM = 128
MSG_DIM = 256
B = 4096

_NC = 2
_NS = 16
_NW = _NC * _NS
_CHUNK = B // _NW

_CP_CHUNK = 3200
_CP_LAST = N_NODES - _CP_CHUNK * (_NW - 1)


def _sc_mesh():
    return plsc.VectorSubcoreMesh(
        core_axis_name="c", subcore_axis_name="s", num_cores=_NC, num_subcores=_NS
    )


def _worker_id():
    return lax.axis_index("s") * _NC + lax.axis_index("c")


def _sc_gather(mem, idx):
    @functools.partial(
        pl.kernel,
        out_type=jax.ShapeDtypeStruct((B, MEM_DIM), jnp.float32),
        mesh=_sc_mesh(),
        scratch_types=[
            pltpu.VMEM((_CHUNK,), jnp.int32),
            pltpu.VMEM((_CHUNK, MEM_DIM), jnp.float32),
            pltpu.SemaphoreType.DMA,
        ],
    )
    def gk(mem_hbm, idx_hbm, out_hbm, idx_v, rows_v, sem):
        base = _worker_id() * _CHUNK
        pltpu.sync_copy(idx_hbm.at[pl.ds(base, _CHUNK)], idx_v)
        pltpu.async_copy(mem_hbm.at[idx_v], rows_v, sem).wait()
        pltpu.sync_copy(rows_v, out_hbm.at[pl.ds(base, _CHUNK)])

    return gk(mem, idx)


_GRU_BLK = 1024


def _gru_body(x_ref, h_ref, wih_ref, whh_ref, bih_ref, bhh_ref, out_ref):
    x = x_ref[...]
    h32 = h_ref[...]
    h = h32.astype(jnp.bfloat16)
    gi = jnp.dot(x, wih_ref[...], preferred_element_type=jnp.float32) + bih_ref[...]
    gh = jnp.dot(h, whh_ref[...], preferred_element_type=jnp.float32) + bhh_ref[...]
    i_r, i_z, i_n = gi[:, :MEM_DIM], gi[:, MEM_DIM : 2 * MEM_DIM], gi[:, 2 * MEM_DIM :]
    h_r, h_z, h_n = gh[:, :MEM_DIM], gh[:, MEM_DIM : 2 * MEM_DIM], gh[:, 2 * MEM_DIM :]
    r = jax.nn.sigmoid(i_r + h_r)
    z = jax.nn.sigmoid(i_z + h_z)
    n = jnp.tanh(i_n + r * h_n)
    out_ref[...] = n + z * (h32 - n)


def _tc_gru(x, h, W_ih, W_hh, b_ih, b_hh):
    x16 = x.astype(jnp.bfloat16)
    wih_t = W_ih.T.astype(jnp.bfloat16)
    whh_t = W_hh.T.astype(jnp.bfloat16)
    bih = b_ih.reshape(1, -1)
    bhh = b_hh.reshape(1, -1)
    grid = B // _GRU_BLK
    return pl.pallas_call(
        _gru_body,
        grid=(grid,),
        in_specs=[
            pl.BlockSpec((_GRU_BLK, MSG_DIM), lambda i: (i, 0)),
            pl.BlockSpec((_GRU_BLK, MEM_DIM), lambda i: (i, 0)),
            pl.BlockSpec((MSG_DIM, 3 * MEM_DIM), lambda i: (0, 0)),
            pl.BlockSpec((MEM_DIM, 3 * MEM_DIM), lambda i: (0, 0)),
            pl.BlockSpec((1, 3 * MEM_DIM), lambda i: (0, 0)),
            pl.BlockSpec((1, 3 * MEM_DIM), lambda i: (0, 0)),
        ],
        out_specs=pl.BlockSpec((_GRU_BLK, MEM_DIM), lambda i: (i, 0)),
        out_shape=jax.ShapeDtypeStruct((B, MEM_DIM), jnp.float32),
    )(x16, h, wih_t, whh_t, bih, bhh)


_JL_CHUNK = 512
_JL_IBLK = 1024


def _jlast_body(nlane_ref, nbcast_ref, out_ref):
    ni = nlane_ref[0]
    nrows = _JL_IBLK // 128
    bests = [jnp.full((1, 128), -1, jnp.int32) for _ in range(nrows)]
    for c in range(B // _JL_CHUNK):
        nj = nbcast_ref[pl.ds(c * _JL_CHUNK, _JL_CHUNK), :]
        jv = lax.broadcasted_iota(jnp.int32, (_JL_CHUNK, 128), 0) + c * _JL_CHUNK
        for r in range(nrows):
            m = jnp.where(nj == ni[r : r + 1, :], jv, -1)
            bests[r] = jnp.maximum(bests[r], jnp.max(m, axis=0, keepdims=True))
    out_ref[0] = jnp.concatenate(bests, axis=0)


def _tc_jlast(nodes):
    nlane = nodes.reshape(B // _JL_IBLK, _JL_IBLK // 128, 128)
    nbcast = jnp.broadcast_to(nodes.reshape(B, 1), (B, 128))
    out = pl.pallas_call(
        _jlast_body,
        grid=(B // _JL_IBLK,),
        in_specs=[
            pl.BlockSpec((1, _JL_IBLK // 128, 128), lambda i: (i, 0, 0)),
            pl.BlockSpec((B, 128), lambda i: (0, 0)),
        ],
        out_specs=pl.BlockSpec((1, _JL_IBLK // 128, 128), lambda i: (i, 0, 0)),
        out_shape=jax.ShapeDtypeStruct((B // _JL_IBLK, _JL_IBLK // 128, 128), jnp.int32),
    )(nlane, nbcast)
    return out.reshape(B)


def _sc_scatter(new_h, j_last, idx, ts, mem_ref, lu_ref):
    @functools.partial(
        pl.kernel,
        out_type=(),
        mesh=_sc_mesh(),
        scratch_types=[
            pltpu.VMEM((_CHUNK,), jnp.int32),
            pltpu.VMEM((_CHUNK,), jnp.int32),
            pltpu.VMEM((_CHUNK, MEM_DIM), jnp.float32),
            pltpu.VMEM((_CHUNK,), jnp.float32),
            pltpu.SemaphoreType.DMA,
            pltpu.SemaphoreType.DMA,
        ],
    )
    def sk(newh_hbm, jl_hbm, idx_hbm, ts_hbm, outmem_hbm, outlu_hbm,
           jl_v, idx_v, rows_v, ts_v, sem1, sem2):
        base = _worker_id() * _CHUNK
        pltpu.sync_copy(jl_hbm.at[pl.ds(base, _CHUNK)], jl_v)
        pltpu.sync_copy(idx_hbm.at[pl.ds(base, _CHUNK)], idx_v)
        g1 = pltpu.async_copy(newh_hbm.at[jl_v], rows_v, sem1)
        g2 = pltpu.async_copy(ts_hbm.at[jl_v], ts_v, sem2)
        g1.wait()
        g2.wait()
        s1 = pltpu.async_copy(rows_v, outmem_hbm.at[idx_v], sem1)
        s2 = pltpu.async_copy(ts_v, outlu_hbm.at[idx_v], sem2)
        s1.wait()
        s2.wait()

    sk(new_h, j_last, idx, ts, mem_ref, lu_ref)


def kernel(memory_tensor, last_update, unique_nodes, unique_messages, unique_ts, W_ih, W_hh, b_ih, b_hh):
    h = _sc_gather(memory_tensor, unique_nodes)
    j_last = _tc_jlast(unique_nodes)
    new_h = _tc_gru(unique_messages, h, W_ih, W_hh, b_ih, b_hh)
    mem_ref = jax.new_ref(memory_tensor)
    lu_ref = jax.new_ref(last_update)
    _sc_scatter(new_h, j_last, unique_nodes, unique_ts, mem_ref, lu_ref)
    return mem_ref[...], lu_ref[...]
